# baseline (speedup 1.0000x reference)
import functools

import jax
import jax.numpy as jnp
from jax import lax
from jax.experimental import pallas as pl
from jax.experimental.pallas import tpu as pltpu
from jax.experimental.pallas import tpu_sc as plsc

NC = 2
NS = 16
L = 16
NW = NC * NS

CHUNK = 32768

_GATHER_DNUMS = lax.GatherDimensionNumbers(
    offset_dims=(), collapsed_slice_dims=(0,), start_index_map=(0,)
)


def _vgather(table, idx):
    return lax.gather(
        table,
        idx[:, None],
        _GATHER_DNUMS,
        slice_sizes=(1,),
        mode=lax.GatherScatterMode.PROMISE_IN_BOUNDS,
    )


def _body(x_hbm, g_hbm, c_hbm, s_hbm, out_hbm, xb, gb, cv, iv):
    n = x_hbm.shape[0]
    per_w = n // NW
    chunk = CHUNK if per_w >= CHUNK else per_w
    nchunk = per_w // chunk
    nvec = chunk // L

    wid = lax.axis_index("s") * NC + lax.axis_index("c")
    base = wid * per_w

    pltpu.sync_copy(c_hbm, cv)
    pltpu.sync_copy(s_hbm, iv)
    cvec = cv[...]
    avec = 1.0 / iv[...]

    def chunk_body(k, _):
        off = base + k * chunk
        pltpu.sync_copy(x_hbm.at[pl.ds(off, chunk)], xb)
        pltpu.sync_copy(g_hbm.at[pl.ds(off, chunk)], gb)

        def vec_body(i, _):
            j = pl.multiple_of(i * L, L)
            idx = gb[pl.ds(j, L)] - 1
            c = _vgather(cvec, idx)
            a = _vgather(avec, idx)
            xb[pl.ds(j, L)] = (xb[pl.ds(j, L)] - c) * a
            return 0

        lax.fori_loop(0, nvec, vec_body, 0)
        pltpu.sync_copy(xb, out_hbm.at[pl.ds(off, chunk)])
        return 0

    lax.fori_loop(0, nchunk, chunk_body, 0)


def kernel(x, group, centers, scales):
    n = x.shape[0]
    chunk = CHUNK if n // NW >= CHUNK else n // NW
    run = pl.kernel(
        _body,
        out_type=jax.ShapeDtypeStruct((n,), jnp.float32),
        mesh=plsc.VectorSubcoreMesh(core_axis_name="c", subcore_axis_name="s"),
        scratch_types=[
            pltpu.VMEM((chunk,), jnp.float32),
            pltpu.VMEM((chunk,), jnp.int32),
            pltpu.VMEM((L,), jnp.float32),
            pltpu.VMEM((L,), jnp.float32),
        ],
    )
    return run(x, group, centers, scales)


# speedup vs baseline: 1.1453x; 1.1453x over previous
import functools

import jax
import jax.numpy as jnp
from jax import lax
from jax.experimental import pallas as pl
from jax.experimental.pallas import tpu as pltpu
from jax.experimental.pallas import tpu_sc as plsc

NC = 2
NS = 16
L = 16
NW = NC * NS

CHUNK = 16384

_GATHER_DNUMS = lax.GatherDimensionNumbers(
    offset_dims=(), collapsed_slice_dims=(0,), start_index_map=(0,)
)


def _vgather(table, idx):
    return lax.gather(
        table,
        idx[:, None],
        _GATHER_DNUMS,
        slice_sizes=(1,),
        mode=lax.GatherScatterMode.PROMISE_IN_BOUNDS,
    )


def _body(x_hbm, g_hbm, c_hbm, s_hbm, out_hbm,
          xb0, xb1, gb0, gb1, cv, iv,
          sin0, sin1, sout0, sout1):
    n = x_hbm.shape[0]
    per_w = n // NW
    chunk = CHUNK if per_w >= CHUNK else per_w
    nchunk = per_w // chunk
    nvec = chunk // L

    wid = lax.axis_index("s") * NC + lax.axis_index("c")
    base = wid * per_w

    pltpu.sync_copy(c_hbm, cv)
    pltpu.sync_copy(s_hbm, iv)
    cvec = cv[...]
    avec = 1.0 / iv[...]

    xbufs = (xb0, xb1)
    gbufs = (gb0, gb1)
    sins = (sin0, sin1)
    souts = (sout0, sout1)

    def start_loads(k):
        b = k % 2
        off = base + k * chunk
        dx = pltpu.async_copy(x_hbm.at[pl.ds(off, chunk)], xbufs[b], sins[b])
        dg = pltpu.async_copy(g_hbm.at[pl.ds(off, chunk)], gbufs[b], sins[b])
        return dx, dg

    loads = {0: start_loads(0)}
    stores = {}
    for k in range(nchunk):
        b = k % 2
        if k + 1 < nchunk:
            if k - 1 >= 0:
                stores.pop(k - 1).wait()
            loads[k + 1] = start_loads(k + 1)
        dx, dg = loads.pop(k)
        dx.wait()
        dg.wait()

        xbuf = xbufs[b]
        gbuf = gbufs[b]

        def vec_body(i, _):
            j = pl.multiple_of(i * L, L)
            idx = gbuf[pl.ds(j, L)] - 1
            c = _vgather(cvec, idx)
            a = _vgather(avec, idx)
            xbuf[pl.ds(j, L)] = (xbuf[pl.ds(j, L)] - c) * a
            return 0

        lax.fori_loop(0, nvec, vec_body, 0)

        off = base + k * chunk
        stores[k] = pltpu.async_copy(
            xbuf, out_hbm.at[pl.ds(off, chunk)], souts[b])
    for k in sorted(stores):
        stores.pop(k).wait()


def kernel(x, group, centers, scales):
    n = x.shape[0]
    chunk = CHUNK if n // NW >= CHUNK else n // NW
    run = pl.kernel(
        _body,
        out_type=jax.ShapeDtypeStruct((n,), jnp.float32),
        mesh=plsc.VectorSubcoreMesh(core_axis_name="c", subcore_axis_name="s"),
        scratch_types=[
            pltpu.VMEM((chunk,), jnp.float32),
            pltpu.VMEM((chunk,), jnp.float32),
            pltpu.VMEM((chunk,), jnp.int32),
            pltpu.VMEM((chunk,), jnp.int32),
            pltpu.VMEM((L,), jnp.float32),
            pltpu.VMEM((L,), jnp.float32),
            pltpu.SemaphoreType.DMA,
            pltpu.SemaphoreType.DMA,
            pltpu.SemaphoreType.DMA,
            pltpu.SemaphoreType.DMA,
        ],
    )
    return run(x, group, centers, scales)


# speedup vs baseline: 2.1859x; 1.9086x over previous
import functools

import jax
import jax.numpy as jnp
from jax import lax
from jax.experimental import pallas as pl
from jax.experimental.pallas import tpu as pltpu
from jax.experimental.pallas import tpu_sc as plsc

NC = 2
NS = 16
L = 16
NW = NC * NS

CHUNK = 16384

_GATHER_DNUMS = lax.GatherDimensionNumbers(
    offset_dims=(), collapsed_slice_dims=(0,), start_index_map=(0,)
)


def _vgather(table, idx):
    return lax.gather(
        table,
        idx[:, None],
        _GATHER_DNUMS,
        slice_sizes=(1,),
        mode=lax.GatherScatterMode.PROMISE_IN_BOUNDS,
    )


def _body(x_hbm, g_hbm, c_hbm, s_hbm, out_hbm,
          xb0, xb1, gb0, gb1, ob0, ob1, cv, iv,
          sin0, sin1, sout0, sout1):
    n = x_hbm.shape[0]
    per_w = n // NW
    chunk = CHUNK if per_w >= CHUNK else per_w
    nchunk = per_w // chunk
    nvec = chunk // L

    wid = lax.axis_index("s") * NC + lax.axis_index("c")
    base = wid * per_w

    pltpu.sync_copy(c_hbm, cv)
    pltpu.sync_copy(s_hbm, iv)
    cvec = cv[...]
    avec = 1.0 / iv[...]

    xbufs = (xb0, xb1)
    gbufs = (gb0, gb1)
    obufs = (ob0, ob1)
    sins = (sin0, sin1)
    souts = (sout0, sout1)

    def start_loads(k):
        b = k % 2
        off = base + k * chunk
        dx = pltpu.async_copy(x_hbm.at[pl.ds(off, chunk)], xbufs[b], sins[b])
        dg = pltpu.async_copy(g_hbm.at[pl.ds(off, chunk)], gbufs[b], sins[b])
        return dx, dg

    loads = {0: start_loads(0)}
    stores = {}
    for k in range(nchunk):
        b = k % 2
        if k + 1 < nchunk:
            if k - 1 >= 0:
                stores.pop(k - 1).wait()
            loads[k + 1] = start_loads(k + 1)
        dx, dg = loads.pop(k)
        dx.wait()
        dg.wait()

        xbuf = xbufs[b]
        gbuf = gbufs[b]
        obuf = obufs[b]

        @plsc.parallel_loop(0, nvec, unroll=8)
        def _(i):
            j = pl.multiple_of(i * L, L)
            idx = gbuf[pl.ds(j, L)] - 1
            c = _vgather(cvec, idx)
            a = _vgather(avec, idx)
            obuf[pl.ds(j, L)] = (xbuf[pl.ds(j, L)] - c) * a

        off = base + k * chunk
        stores[k] = pltpu.async_copy(
            obuf, out_hbm.at[pl.ds(off, chunk)], souts[b])
    for k in sorted(stores):
        stores.pop(k).wait()


def kernel(x, group, centers, scales):
    n = x.shape[0]
    chunk = CHUNK if n // NW >= CHUNK else n // NW
    run = pl.kernel(
        _body,
        out_type=jax.ShapeDtypeStruct((n,), jnp.float32),
        mesh=plsc.VectorSubcoreMesh(core_axis_name="c", subcore_axis_name="s"),
        scratch_types=[
            pltpu.VMEM((chunk,), jnp.float32),
            pltpu.VMEM((chunk,), jnp.float32),
            pltpu.VMEM((chunk,), jnp.int32),
            pltpu.VMEM((chunk,), jnp.int32),
            pltpu.VMEM((chunk,), jnp.float32),
            pltpu.VMEM((chunk,), jnp.float32),
            pltpu.VMEM((L,), jnp.float32),
            pltpu.VMEM((L,), jnp.float32),
            pltpu.SemaphoreType.DMA,
            pltpu.SemaphoreType.DMA,
            pltpu.SemaphoreType.DMA,
            pltpu.SemaphoreType.DMA,
        ],
    )
    return run(x, group, centers, scales)


# speedup vs baseline: 2.2325x; 1.0213x over previous
import jax
import jax.numpy as jnp
from jax import lax
from jax.experimental import pallas as pl
from jax.experimental.pallas import tpu as pltpu
from jax.experimental.pallas import tpu_sc as plsc

NC = 2
NS = 16
L = 16
NW = NC * NS

CHUNK = 16384
NBUF = 3

_GATHER_DNUMS = lax.GatherDimensionNumbers(
    offset_dims=(), collapsed_slice_dims=(0,), start_index_map=(0,)
)


def _vgather(table, idx):
    return lax.gather(
        table,
        idx[:, None],
        _GATHER_DNUMS,
        slice_sizes=(1,),
        mode=lax.GatherScatterMode.PROMISE_IN_BOUNDS,
    )


def _body(x_hbm, g_hbm, c_hbm, s_hbm, out_hbm,
          xb0, xb1, xb2, gb0, gb1, gb2, cv, iv,
          sin0, sin1, sin2, sout0, sout1, sout2):
    n = x_hbm.shape[0]
    per_w = n // NW
    chunk = CHUNK if per_w >= CHUNK else per_w
    nchunk = per_w // chunk
    nvec = chunk // L

    wid = lax.axis_index("s") * NC + lax.axis_index("c")
    base = wid * per_w

    pltpu.sync_copy(c_hbm, cv)
    pltpu.sync_copy(s_hbm, iv)
    cvec = cv[...]
    avec = 1.0 / iv[...]

    xbufs = (xb0, xb1, xb2)
    gbufs = (gb0, gb1, gb2)
    sins = (sin0, sin1, sin2)
    souts = (sout0, sout1, sout2)
    nb = min(NBUF, nchunk)

    def start_loads(k):
        b = k % nb
        off = base + k * chunk
        dx = pltpu.async_copy(x_hbm.at[pl.ds(off, chunk)], xbufs[b], sins[b])
        dg = pltpu.async_copy(g_hbm.at[pl.ds(off, chunk)], gbufs[b], sins[b])
        return dx, dg

    loads = {}
    stores = {}
    for k in range(min(nb - 1, nchunk)):
        loads[k] = start_loads(k)
    for k in range(nchunk):
        b = k % nb
        if k + nb - 1 < nchunk:
            if k - 1 >= 0:
                stores.pop(k - 1).wait()
            loads[k + nb - 1] = start_loads(k + nb - 1)
        dx, dg = loads.pop(k)
        dx.wait()
        dg.wait()

        xbuf = xbufs[b]
        gbuf = gbufs[b]

        @plsc.parallel_loop(0, nvec, unroll=8)
        def _(i):
            j = pl.multiple_of(i * L, L)
            idx = gbuf[pl.ds(j, L)] - 1
            c = _vgather(cvec, idx)
            a = _vgather(avec, idx)
            xbuf[pl.ds(j, L)] = (xbuf[pl.ds(j, L)] - c) * a

        off = base + k * chunk
        stores[k] = pltpu.async_copy(
            xbuf, out_hbm.at[pl.ds(off, chunk)], souts[b])
    for k in sorted(stores):
        stores.pop(k).wait()


def kernel(x, group, centers, scales):
    n = x.shape[0]
    chunk = CHUNK if n // NW >= CHUNK else n // NW
    run = pl.kernel(
        _body,
        out_type=jax.ShapeDtypeStruct((n,), jnp.float32),
        mesh=plsc.VectorSubcoreMesh(core_axis_name="c", subcore_axis_name="s"),
        scratch_types=[
            pltpu.VMEM((chunk,), jnp.float32),
            pltpu.VMEM((chunk,), jnp.float32),
            pltpu.VMEM((chunk,), jnp.float32),
            pltpu.VMEM((chunk,), jnp.int32),
            pltpu.VMEM((chunk,), jnp.int32),
            pltpu.VMEM((chunk,), jnp.int32),
            pltpu.VMEM((L,), jnp.float32),
            pltpu.VMEM((L,), jnp.float32),
            pltpu.SemaphoreType.DMA,
            pltpu.SemaphoreType.DMA,
            pltpu.SemaphoreType.DMA,
            pltpu.SemaphoreType.DMA,
            pltpu.SemaphoreType.DMA,
            pltpu.SemaphoreType.DMA,
        ],
    )
    return run(x, group, centers, scales)
